# Initial kernel scaffold; baseline (speedup 1.0000x reference)
import functools

import jax
import jax.numpy as jnp
from jax import lax
from jax.experimental import pallas as pl
from jax.experimental.pallas import tpu as pltpu
from jax.experimental.pallas import tpu_sc as plsc

_V = 1401
_PAD = 1400
_D = 128
_S = 20
_N = 4096 * 50
_NC = 2
_NS = 16
_L = 16
_DH = _D // _NC
_NPOS = _N // _NS
_CH = 128
_NCHUNK = _NPOS // _CH


def _tree_sum(vals):
    vals = list(vals)
    while len(vals) > 1:
        nxt = [vals[i] + vals[i + 1] for i in range(0, len(vals) - 1, 2)]
        if len(vals) % 2:
            nxt.append(vals[-1])
        vals = nxt
    return vals[0]


def _scale_body(t_ref, o_ref):
    t = t_ref[...]
    norm = jnp.sqrt(jnp.sum(t * t, axis=1, keepdims=True))
    scale = jnp.minimum(1.0, 5.0 / jnp.maximum(norm, 1e-7))
    o_ref[...] = t * scale


def _scale_table(table):
    return pl.pallas_call(
        _scale_body,
        out_shape=jax.ShapeDtypeStruct((_V, _D), jnp.float32),
    )(table)


def _sc_body(x_hbm, tab_hbm, out_hbm, tab_v, idx_v, out_v):
    c = lax.axis_index("c")
    s = lax.axis_index("s")
    dbase = c * _DH
    pbase = s * _NPOS
    pltpu.sync_copy(tab_hbm.at[:, pl.ds(dbase, _DH)], tab_v)
    iota = lax.iota(jnp.int32, _L)

    def chunk_body(ci, carry):
        base = pbase + ci * _CH
        pltpu.sync_copy(x_hbm.at[pl.ds(base, _CH), :], idx_v)

        def group_body(g, carry2):
            p = g * _L + iota
            rows = [
                plsc.load_gather(idx_v, [p, jnp.full((_L,), si, jnp.int32)])
                for si in range(_S)
            ]
            cnt = _tree_sum(
                [jnp.where(r < _PAD, 1.0, 0.0).astype(jnp.float32) for r in rows]
            )
            recip = 1.0 / jnp.maximum(cnt, 1.0)
            for d in range(_DH):
                dcol = jnp.full((_L,), d, jnp.int32)
                acc = _tree_sum([plsc.load_gather(tab_v, [r, dcol]) for r in rows])
                plsc.store_scatter(out_v, [p, dcol], acc * recip)
            return carry2

        lax.fori_loop(0, _CH // _L, group_body, 0)
        pltpu.sync_copy(out_v, out_hbm.at[pl.ds(base, _CH), pl.ds(dbase, _DH)])
        return carry

    lax.fori_loop(0, _NCHUNK, chunk_body, 0)


@functools.partial(
    pl.kernel,
    out_type=jax.ShapeDtypeStruct((_N, _D), jnp.float32),
    mesh=plsc.VectorSubcoreMesh(
        core_axis_name="c", subcore_axis_name="s", num_cores=_NC, num_subcores=_NS
    ),
    scratch_types=[
        pltpu.VMEM((_V, _DH), jnp.float32),
        pltpu.VMEM((_CH, _S), jnp.int32),
        pltpu.VMEM((_CH, _DH), jnp.float32),
    ],
)
def _sc_lookup(x_hbm, tab_hbm, out_hbm, tab_v, idx_v, out_v):
    _sc_body(x_hbm, tab_hbm, out_hbm, tab_v, idx_v, out_v)


def kernel(x, table):
    b, l, s = x.shape
    scaled = _scale_table(table.astype(jnp.float32))
    out = _sc_lookup(x.reshape(_N, _S), scaled)
    return out.reshape(b, l, _D)


# speedup vs baseline: 1.9529x; 1.9529x over previous
import functools

import jax
import jax.numpy as jnp
from jax import lax
from jax.experimental import pallas as pl
from jax.experimental.pallas import tpu as pltpu
from jax.experimental.pallas import tpu_sc as plsc

_V = 1401
_PAD = 1400
_D = 128
_S = 20
_N = 4096 * 50
_NC = 2
_NS = 16
_L = 16
_DH = _D // _NC
_NPOS = _N // _NS
_CH = 128
_NCHUNK = _NPOS // _CH


def _tree_sum(vals):
    vals = list(vals)
    while len(vals) > 1:
        nxt = [vals[i] + vals[i + 1] for i in range(0, len(vals) - 1, 2)]
        if len(vals) % 2:
            nxt.append(vals[-1])
        vals = nxt
    return vals[0]


def _scale_body(t_ref, o_ref):
    t = t_ref[...]
    norm = jnp.sqrt(jnp.sum(t * t, axis=1, keepdims=True))
    scale = jnp.minimum(1.0, 5.0 / jnp.maximum(norm, 1e-7))
    o_ref[...] = t * scale


def _scale_table(table):
    return pl.pallas_call(
        _scale_body,
        out_shape=jax.ShapeDtypeStruct((_V, _D), jnp.float32),
    )(table)


def _sc_body(x_hbm, tab_hbm, out_hbm, tab_v, idx_v, out_v):
    c = lax.axis_index("c")
    s = lax.axis_index("s")
    dbase = c * _DH
    pbase = s * _NPOS
    pltpu.sync_copy(tab_hbm.at[:, pl.ds(dbase, _DH)], tab_v)
    iota = lax.iota(jnp.int32, _L)

    def chunk_body(ci, carry):
        base = pbase + ci * _CH
        pltpu.sync_copy(x_hbm.at[pl.ds(base, _CH), :], idx_v)

        def group_body(g, carry2):
            p = g * _L + iota
            rows = [
                plsc.load_gather(idx_v, [p, jnp.full((_L,), si, jnp.int32)])
                for si in range(_S)
            ]
            cnt = _tree_sum(
                [jnp.where(r < _PAD, 1.0, 0.0).astype(jnp.float32) for r in rows]
            )
            recip = 1.0 / jnp.maximum(cnt, 1.0)
            for d in range(_DH):
                dcol = jnp.full((_L,), d, jnp.int32)
                acc = _tree_sum([plsc.load_gather(tab_v, [r, dcol]) for r in rows])
                plsc.store_scatter(out_v, [p, dcol], acc * recip)
            return carry2

        lax.fori_loop(0, _CH // _L, group_body, 0)
        pltpu.sync_copy(out_v, out_hbm.at[pl.ds(base, _CH), pl.ds(dbase, _DH)])
        return carry

    lax.fori_loop(0, _NCHUNK, chunk_body, 0)


@functools.partial(
    pl.kernel,
    out_type=jax.ShapeDtypeStruct((_N, _D), jnp.float32),
    compiler_params=pltpu.CompilerParams(use_tc_tiling_on_sc=False, needs_layout_passes=False),
    mesh=plsc.VectorSubcoreMesh(
        core_axis_name="c", subcore_axis_name="s", num_cores=_NC, num_subcores=_NS
    ),
    scratch_types=[
        pltpu.VMEM((_V, _DH), jnp.float32),
        pltpu.VMEM((_CH, _S), jnp.int32),
        pltpu.VMEM((_CH, _DH), jnp.float32),
    ],
)
def _sc_lookup(x_hbm, tab_hbm, out_hbm, tab_v, idx_v, out_v):
    _sc_body(x_hbm, tab_hbm, out_hbm, tab_v, idx_v, out_v)


def kernel(x, table):
    b, l, s = x.shape
    scaled = _scale_table(table.astype(jnp.float32))
    out = _sc_lookup(x.reshape(_N, _S), scaled)
    return out.reshape(b, l, _D)


# speedup vs baseline: 5.2042x; 2.6648x over previous
import functools

import jax
import jax.numpy as jnp
from jax import lax
from jax.experimental import pallas as pl
from jax.experimental.pallas import tpu as pltpu
from jax.experimental.pallas import tpu_sc as plsc

_V = 1401
_PAD = 1400
_D = 128
_S = 20
_N = 4096 * 50
_NC = 2
_NS = 16
_L = 16
_NW = _NC * _NS
_NPOS = _N // _NW
_CH = 32
_ROWS = _CH * _S
_RPS = 80
_NSTR = _ROWS // _RPS
_NCHUNK = _NPOS // _CH


def _scale_body(t_ref, o_ref):
    t = t_ref[...]
    norm = jnp.sqrt(jnp.sum(t * t, axis=1, keepdims=True))
    scale = jnp.minimum(1.0, 5.0 / jnp.maximum(norm, 1e-7))
    o_ref[...] = t * scale


def _scale_table(table):
    return pl.pallas_call(
        _scale_body,
        out_shape=jax.ShapeDtypeStruct((_V, _D), jnp.float32),
    )(table)


def _recip_body(x_ref, r_ref):
    cnt = jnp.sum((x_ref[...] < _PAD).astype(jnp.float32), axis=1)
    r_ref[...] = 1.0 / jnp.maximum(cnt, 1.0)


def _recip_counts(x2d):
    nb = 8192
    return pl.pallas_call(
        _recip_body,
        grid=(_N // nb,),
        in_specs=[pl.BlockSpec((nb, _S), lambda i: (i, 0))],
        out_specs=pl.BlockSpec((nb,), lambda i: (i,)),
        out_shape=jax.ShapeDtypeStruct((_N,), jnp.float32),
    )(x2d)


def _sc_body(xf_hbm, tab_hbm, rcp_hbm, out_hbm,
             rows_v, idx_v, out_v, zeros_v, didx_v, pidx_v, rcp_v, acc_sh,
             gsem, ssem):
    c = lax.axis_index("c")
    s = lax.axis_index("s")
    sid = s
    wid = s * _NC + c
    pbase = wid * _NPOS
    iota = lax.iota(jnp.int32, _L)

    def init_body(i, carry):
        p = i // 8
        k = i % 8
        zeros_v[p, pl.ds(k * _L, _L)] = jnp.zeros((_L,), jnp.float32)
        return carry

    lax.fori_loop(0, _CH * 8, init_body, 0)
    pltpu.sync_copy(zeros_v, acc_sh.at[sid, 0, pl.ds(0, _CH)])
    pltpu.sync_copy(zeros_v, acc_sh.at[sid, 1, pl.ds(0, _CH)])
    pidx_v[0, pl.ds(0, _L)] = jnp.full((_L,), _CH, jnp.int32)
    for j in range(_NSTR):
        for g in range(_RPS // _L):
            flat = j * _RPS + g * _L + iota
            didx_v[j, pl.ds(g * _L, _L)] = flat // _S

    def chunk_body(ci, carry):
        base = pbase + ci * _CH
        b20 = base * _S
        for j in range(_NSTR):
            pltpu.sync_copy(xf_hbm.at[pl.ds(b20 + j * _RPS, _RPS)], idx_v.at[j])
        pltpu.sync_copy(rcp_hbm.at[pl.ds(base, _CH)], rcp_v)
        bi = lax.rem(ci, 2)
        for j in range(_NSTR):
            pltpu.async_copy(
                tab_hbm.at[idx_v.at[j]], rows_v.at[pl.ds(j * _RPS, _RPS)], gsem
            ).wait()
        pltpu.async_copy(
            zeros_v.at[pl.ds(0, _L)],
            acc_sh.at[sid, bi].at[pidx_v.at[0]],
            ssem,
            add=True,
        ).wait()
        for j in range(_NSTR):
            pltpu.async_copy(
                rows_v.at[pl.ds(j * _RPS, _RPS)],
                acc_sh.at[sid, bi].at[didx_v.at[j]],
                ssem,
                add=True,
            ).wait()
        pltpu.sync_copy(acc_sh.at[sid, bi, pl.ds(1, _CH - 1)],
                        out_v.at[pl.ds(1, _CH - 1)])
        pltpu.sync_copy(zeros_v, acc_sh.at[sid, bi, pl.ds(0, _CH)])
        v0 = idx_v[0, pl.ds(0, _L)]
        v1 = idx_v[0, pl.ds(_L, _L)]
        ones = jnp.ones((_L,), jnp.float32)
        zs = jnp.zeros((_L,), jnp.float32)
        c0 = jnp.where(v0 < _PAD, ones, zs)
        c1 = jnp.where(jnp.logical_and(iota < _S - _L, v1 < _PAD), ones, zs)
        cnt0 = lax.reduce_sum_p.bind(c0 + c1, axes=(0,))
        rv0 = jnp.full((_L,), 1.0, jnp.float32) / jnp.maximum(
            jnp.full((_L,), cnt0, jnp.float32), 1.0)
        for k in range(_D // _L):
            vals = [rows_v[r, pl.ds(k * _L, _L)] for r in range(_S)]
            while len(vals) > 1:
                nxt = [vals[i] + vals[i + 1] for i in range(0, len(vals) - 1, 2)]
                if len(vals) % 2:
                    nxt.append(vals[-1])
                vals = nxt
            out_v[0, pl.ds(k * _L, _L)] = vals[0] * rv0
        for p in range(1, _CH):
            rv = plsc.load_gather(rcp_v, [jnp.full((_L,), p, jnp.int32)])
            for k in range(_D // _L):
                out_v[p, pl.ds(k * _L, _L)] = out_v[p, pl.ds(k * _L, _L)] * rv
        pltpu.sync_copy(out_v, out_hbm.at[pl.ds(base, _CH)])
        return carry

    lax.fori_loop(0, _NCHUNK, chunk_body, 0)


@functools.partial(
    pl.kernel,
    out_type=jax.ShapeDtypeStruct((_N, _D), jnp.float32),
    compiler_params=pltpu.CompilerParams(
        use_tc_tiling_on_sc=False, needs_layout_passes=False
    ),
    mesh=plsc.VectorSubcoreMesh(
        core_axis_name="c", subcore_axis_name="s", num_cores=_NC, num_subcores=_NS
    ),
    scratch_types=[
        pltpu.VMEM((_ROWS, _D), jnp.float32),
        pltpu.VMEM((_NSTR, _RPS), jnp.int32),
        pltpu.VMEM((_CH, _D), jnp.float32),
        pltpu.VMEM((_CH, _D), jnp.float32),
        pltpu.VMEM((_NSTR, _RPS), jnp.int32),
        pltpu.VMEM((1, _L), jnp.int32),
        pltpu.VMEM((_CH,), jnp.float32),
        pltpu.VMEM_SHARED((_NS, 2, _CH + 1, _D), jnp.float32),
        pltpu.SemaphoreType.DMA,
        pltpu.SemaphoreType.DMA,
    ],
)
def _sc_lookup(xf_hbm, tab_hbm, rcp_hbm, out_hbm,
               rows_v, idx_v, out_v, zeros_v, didx_v, pidx_v, rcp_v, acc_sh,
               gsem, ssem):
    _sc_body(xf_hbm, tab_hbm, rcp_hbm, out_hbm,
             rows_v, idx_v, out_v, zeros_v, didx_v, pidx_v, rcp_v, acc_sh,
             gsem, ssem)


def kernel(x, table):
    b, l, s = x.shape
    x2d = x.reshape(_N, _S)
    scaled = _scale_table(table.astype(jnp.float32))
    rcp = _recip_counts(x2d)
    out = _sc_lookup(x2d.reshape(_N * _S), scaled, rcp)
    return out.reshape(b, l, _D)
